# baseline (speedup 1.0000x reference)
import functools

import jax
import jax.numpy as jnp
from jax import lax
from jax.experimental import pallas as pl
from jax.experimental.pallas import tpu as pltpu
from jax.experimental.pallas import tpu_sc as plsc

B = 16384
D = 64
NW = 32
BPW = B // NW
IDX_MINOR = 128
NCHUNK = BPW // IDX_MINOR
NTOT = B * D


def _sc_gather_stats(c2d, noise, table, lw16, ct16):
    mesh = plsc.VectorSubcoreMesh(core_axis_name="c", subcore_axis_name="s")

    @functools.partial(
        pl.kernel,
        mesh=mesh,
        out_type=[
            jax.ShapeDtypeStruct((B, D), jnp.float32),
            jax.ShapeDtypeStruct((2 * NW, 16), jnp.float32),
        ],
        scratch_types=[
            pltpu.VMEM((NCHUNK, IDX_MINOR), jnp.int32),
            pltpu.VMEM((BPW, D), jnp.float32),
            pltpu.VMEM((BPW, D), jnp.float32),
            pltpu.VMEM((16,), jnp.float32),
            pltpu.VMEM((16,), jnp.int32),
            pltpu.VMEM((16,), jnp.float32),
            pltpu.VMEM((16,), jnp.float32),
            pltpu.SemaphoreType.DMA,
            pltpu.SemaphoreType.DMA,
        ],
        compiler_params=pltpu.CompilerParams(use_tc_tiling_on_sc=False),
    )
    def k(c_hbm, noise_hbm, table_hbm, lw_hbm, ct_hbm, y_hbm, part_hbm,
          idx_v, rows_v, noise_v, lw_v, ct_v, s_v, q_v, gsem, nsem):
        wid = lax.axis_index("s") * 2 + lax.axis_index("c")
        base = wid * BPW
        pltpu.sync_copy(c_hbm.at[pl.ds(wid * NCHUNK, NCHUNK)], idx_v)
        ncopy = pltpu.async_copy(noise_hbm.at[pl.ds(base, BPW)], noise_v, nsem)
        copies = []
        for j in range(NCHUNK):
            copies.append(pltpu.async_copy(
                table_hbm.at[idx_v.at[j]],
                rows_v.at[pl.ds(j * IDX_MINOR, IDX_MINOR)],
                gsem))
        pltpu.sync_copy(lw_hbm, lw_v)
        pltpu.sync_copy(ct_hbm, ct_v)
        lw = lw_v[...]
        ct = ct_v[...]
        scale = jnp.where(ct != 0, 1.0 / (1.0 + jnp.exp(-lw)), 0.0)
        for cp in copies:
            cp.wait()
        ncopy.wait()

        def body(i, carry):
            s, q = carry
            for j in range(D // 16):
                e = rows_v[i, pl.ds(j * 16, 16)]
                nz = noise_v[i, pl.ds(j * 16, 16)]
                y = e * scale + nz
                rows_v[i, pl.ds(j * 16, 16)] = y
                s = s + y
                q = q + y * y
            return s, q

        zero = jnp.zeros((16,), jnp.float32)
        s, q = lax.fori_loop(0, BPW, body, (zero, zero))
        s_v[...] = s
        q_v[...] = q
        pltpu.sync_copy(rows_v, y_hbm.at[pl.ds(base, BPW)])
        pltpu.sync_copy(s_v, part_hbm.at[wid])
        pltpu.sync_copy(q_v, part_hbm.at[NW + wid])

    return k(c2d, noise, table, lw16, ct16)


def _tc_normalize(y, part8x128):
    blk = 1024
    grid = B // blk

    def body(part_ref, y_ref, o_ref):
        p = part_ref[...]
        s1 = jnp.sum(p[:4, :])
        s2 = jnp.sum(p[4:, :])
        mean = s1 / NTOT
        var = (s2 - s1 * s1 / NTOT) / (NTOT - 1)
        rstd = lax.rsqrt(var)
        o_ref[...] = (y_ref[...] - mean) * rstd

    return pl.pallas_call(
        body,
        grid=(grid,),
        in_specs=[
            pl.BlockSpec((8, 128), lambda i: (0, 0)),
            pl.BlockSpec((blk, D), lambda i: (i, 0)),
        ],
        out_specs=pl.BlockSpec((blk, D), lambda i: (i, 0)),
        out_shape=jax.ShapeDtypeStruct((B, D), jnp.float32),
    )(part8x128, y)


def kernel(noise, c, embed_table, label_weight, c_training):
    c2d = c.reshape(NW * NCHUNK, IDX_MINOR)
    lw16 = jnp.broadcast_to(label_weight.astype(jnp.float32), (16,))
    ct16 = jnp.broadcast_to(jnp.asarray(c_training, jnp.int32), (16,))
    y, part = _sc_gather_stats(c2d, noise, embed_table, lw16, ct16)
    return _tc_normalize(y, part.reshape(8, 128))
